# baseline (speedup 1.0000x reference)
import functools

import jax
import jax.numpy as jnp
from jax.experimental import pallas as pl
from jax.experimental.pallas import tpu as pltpu

_T_IN = 10
_STEP = 10
_MODES = 8
_WIDTH = 32
_P = 2 * _MODES * _MODES
_INV_SQRT2 = 0.7071067811865476
_VMEM_LIMIT = 48 * 1024 * 1024


def _gelu_scaled_bf16(yp):
    return (yp * (1.0 + jax.lax.erf(yp))).astype(jnp.bfloat16)


def _dft_bases(X, Y, S_pad):
    m = _MODES
    kx = jnp.concatenate([jnp.arange(m), jnp.arange(X - m, X)]).astype(jnp.float32)
    ky = jnp.arange(m, dtype=jnp.float32)
    xs = jnp.arange(X, dtype=jnp.float32)
    ys = jnp.arange(Y, dtype=jnp.float32)
    tx = 2.0 * jnp.pi * kx[:, None] * xs[None, :] / X
    ty = 2.0 * jnp.pi * ky[:, None] * ys[None, :] / Y
    theta = (tx[:, None, :, None] + ty[None, :, None, :]).reshape(_P, X * Y)
    ert = jnp.cos(theta).T
    eit = (-jnp.sin(theta)).T
    c = jnp.where(ky == 0, 1.0, 2.0)
    cc = jnp.broadcast_to(c[None, :], (2 * m, m)).reshape(_P, 1)
    dr = cc * jnp.cos(theta) / (X * Y)
    di = cc * jnp.sin(theta) / (X * Y)
    pad = S_pad - X * Y
    if pad:
        ert = jnp.pad(ert, ((0, pad), (0, 0)))
        eit = jnp.pad(eit, ((0, pad), (0, 0)))
        dr = jnp.pad(dr, ((0, 0), (0, pad)))
        di = jnp.pad(di, ((0, 0), (0, pad)))
    ertit = jnp.concatenate([ert, eit], axis=1)
    drdi = jnp.concatenate([dr, -di], axis=0)
    return ertit, drdi


def _pack_spec(w1r, w1i, w2r, w2i):
    def flat(w):
        ci, co, m1, m2 = w.shape
        return w.reshape(ci, co, m1 * m2)
    wr = jnp.concatenate([flat(w1r), flat(w2r)], axis=-1)
    wi = jnp.concatenate([flat(w1i), flat(w2i)], axis=-1)
    return wr, wi


def _fno_kernel(G, C, Sp, xa_ref, wl_ref, ertit_ref, drdi_ref,
                wr0_ref, wi0_ref, ws0_ref, wr1_ref, wi1_ref, ws1_ref,
                wr2_ref, wi2_ref, ws2_ref,
                cwx0_ref, cwx1_ref, cwx2_ref,
                f1w_ref, f2w_ref, f2b_ref, o_ref, hs_ref):
    f32 = jnp.float32
    bf16 = jnp.bfloat16
    R = G * C
    E = R + 8

    @pl.when(pl.program_id(0) == 0)
    def _init():
        band = jax.lax.broadcasted_iota(jnp.int32, (8, Sp), 0) == 0
        hs_ref[R:E, :] = band.astype(bf16)
        hs_ref[E:, :] = drdi_ref[...]

    hs_ref[0:R, :] = jax.lax.dot_general(
        wl_ref[...], xa_ref[0], (((1,), (0,)), ((), ())),
        preferred_element_type=f32).astype(bf16)

    wrs = (wr0_ref, wr1_ref, wr2_ref)
    wis = (wi0_ref, wi1_ref, wi2_ref)
    wss = (ws0_ref, ws1_ref, ws2_ref)
    cwxs = (cwx0_ref, cwx1_ref, cwx2_ref)
    for k in range(3):
        xf = jnp.dot(hs_ref[0:R, :], ertit_ref[...],
                     preferred_element_type=f32)
        xr = xf[:, :_P].reshape(G, C, 1, _P)
        xi = xf[:, _P:].reshape(G, C, 1, _P)
        t1 = jnp.sum(xr * wrs[k][...][None], axis=1)
        t2 = jnp.sum(xi * wis[k][...][None], axis=1)
        t3 = jnp.sum((xr + xi) * wss[k][...][None], axis=1)
        omr = t1 - t2
        omi = t3 - t1 - t2
        om = jnp.concatenate([omr.reshape(R, _P), omi.reshape(R, _P)],
                             axis=1).astype(bf16)
        lhs = jnp.concatenate([cwxs[k][...], om], axis=1)
        y = jnp.dot(lhs, hs_ref[...], preferred_element_type=f32)
        hs_ref[0:R, :] = _gelu_scaled_bf16(y)

    t = jnp.dot(f1w_ref[...], hs_ref[0:E, :], preferred_element_type=f32)
    h2 = _gelu_scaled_bf16(t)
    out = jnp.dot(f2w_ref[...], h2, preferred_element_type=f32)
    o_ref[0] = out + f2b_ref[...]


def kernel(x, fc0_w, fc0_b, conv0_w1r, conv0_w1i, conv0_w2r, conv0_w2i,
           w0_w, w0_b, conv1_w1r, conv1_w1i, conv1_w2r, conv1_w2i, w1_w,
           w1_b, conv2_w1r, conv2_w1i, conv2_w2r, conv2_w2i, w2_w, w2_b,
           fc1_w, fc1_b, fc2_w, fc2_b):
    B, X, Y, Tin = x.shape
    C = _WIDTH
    S = X * Y
    Sp = pl.cdiv(S, 128) * 128
    G = 4 if B % 4 == 0 else (2 if B % 2 == 0 else 1)
    NB = B // G
    f32 = jnp.float32
    bf16 = jnp.bfloat16

    x_cf = jnp.transpose(x, (0, 3, 1, 2)).reshape(B, Tin, S)
    gx = jnp.linspace(-1.5, 1.5, X, dtype=f32)
    gy = jnp.linspace(-2.0, 2.0, Y, dtype=f32)
    grid2 = jnp.stack([jnp.broadcast_to(gx[:, None], (X, Y)).reshape(S),
                       jnp.broadcast_to(gy[None, :], (X, Y)).reshape(S)], 0)
    aug = jnp.concatenate(
        [x_cf,
         jnp.broadcast_to(grid2[None], (B, 2, S)),
         jnp.ones((B, 1, S), f32)], axis=1)
    if Sp != S:
        aug = jnp.pad(aug, ((0, 0), (0, 0), (0, Sp - S)))
    K0 = Tin + 3
    xa = aug.reshape(NB, G * K0, Sp).astype(bf16)

    SQ2 = 1.4142135623730951
    R = G * C
    eyeG = jnp.eye(G, dtype=f32)
    wl = jnp.concatenate([fc0_w.T, fc0_b[:, None]], axis=1) * SQ2
    wl_bd = jnp.kron(eyeG, wl).astype(bf16)

    ertit, drdi = _dft_bases(X, Y, Sp)
    ertit = ertit.astype(bf16)
    drdi = (0.5 * drdi).astype(bf16)

    wr0, wi0 = _pack_spec(conv0_w1r, conv0_w1i, conv0_w2r, conv0_w2i)
    wr1, wi1 = _pack_spec(conv1_w1r, conv1_w1i, conv1_w2r, conv1_w2i)
    wr2, wi2 = _pack_spec(conv2_w1r, conv2_w1i, conv2_w2r, conv2_w2i)
    ws0 = wr0 + wi0
    ws1 = wr1 + wi1
    ws2 = wr2 + wi2

    def conv_ext(w, b):
        bd = jnp.kron(eyeG, 0.5 * w.T)
        col = _INV_SQRT2 * jnp.tile(b, G)[:, None]
        return jnp.concatenate([bd, col, jnp.zeros((R, 7), f32)],
                               axis=1).astype(bf16)

    cwx0 = conv_ext(w0_w, w0_b)
    cwx1 = conv_ext(w1_w, w1_b)
    cwx2 = conv_ext(w2_w, w2_b)

    H = fc1_w.shape[1]
    O = fc2_w.shape[1]
    f1_bd = jnp.kron(eyeG, 0.5 * fc1_w.T)
    f1_col = _INV_SQRT2 * jnp.tile(fc1_b, G)[:, None]
    f1w_ext = jnp.concatenate([f1_bd, f1_col, jnp.zeros((G * H, 7), f32)],
                              axis=1).astype(bf16)
    f2w_bd = jnp.kron(eyeG, _INV_SQRT2 * fc2_w.T).astype(bf16)
    f2b = jnp.tile(fc2_b, G)[:, None]

    const = lambda shp: pl.BlockSpec(shp, lambda i: tuple(0 for _ in shp))
    specP = (C, C, _P)
    out_cf = pl.pallas_call(
        functools.partial(_fno_kernel, G, C, Sp),
        grid=(NB,),
        in_specs=[pl.BlockSpec((1, G * K0, Sp), lambda i: (i, 0, 0)),
                  const((G * C, G * K0)),
                  const((Sp, 2 * _P)), const((2 * _P, Sp)),
                  const(specP), const(specP), const(specP),
                  const(specP), const(specP), const(specP),
                  const(specP), const(specP), const(specP),
                  const((R, R + 8)), const((R, R + 8)), const((R, R + 8)),
                  const((G * H, R + 8)),
                  const((G * O, G * H)), const((G * O, 1))],
        out_specs=pl.BlockSpec((1, G * O, Sp), lambda i: (i, 0, 0)),
        out_shape=jax.ShapeDtypeStruct((NB, G * O, Sp), f32),
        scratch_shapes=[pltpu.VMEM((R + 8 + 2 * _P, Sp), bf16)],
        compiler_params=pltpu.CompilerParams(
            dimension_semantics=("arbitrary",),
            vmem_limit_bytes=_VMEM_LIMIT),
    )(xa, wl_bd, ertit, drdi,
      wr0, wi0, ws0, wr1, wi1, ws1, wr2, wi2, ws2,
      cwx0, cwx1, cwx2, f1w_ext, f2w_bd, f2b)

    out_cf = out_cf.reshape(B, O, Sp)[:, :, :S]
    return jnp.transpose(out_cf.reshape(B, O, X, Y), (0, 2, 3, 1))


# speedup vs baseline: 1.0367x; 1.0367x over previous
import functools

import jax
import jax.numpy as jnp
from jax.experimental import pallas as pl
from jax.experimental.pallas import tpu as pltpu

_T_IN = 10
_STEP = 10
_MODES = 8
_WIDTH = 32
_P = 2 * _MODES * _MODES
_INV_SQRT2 = 0.7071067811865476
_VMEM_LIMIT = 48 * 1024 * 1024


def _gelu_scaled_bf16(yp):
    return (yp * (1.0 + jax.lax.erf(yp))).astype(jnp.bfloat16)


def _dft_bases(X, Y, S_pad):
    m = _MODES
    kx = jnp.concatenate([jnp.arange(m), jnp.arange(X - m, X)]).astype(jnp.float32)
    ky = jnp.arange(m, dtype=jnp.float32)
    xs = jnp.arange(X, dtype=jnp.float32)
    ys = jnp.arange(Y, dtype=jnp.float32)
    tx = 2.0 * jnp.pi * kx[:, None] * xs[None, :] / X
    ty = 2.0 * jnp.pi * ky[:, None] * ys[None, :] / Y
    theta = (tx[:, None, :, None] + ty[None, :, None, :]).reshape(_P, X * Y)
    ert = jnp.cos(theta).T
    eit = (-jnp.sin(theta)).T
    c = jnp.where(ky == 0, 1.0, 2.0)
    cc = jnp.broadcast_to(c[None, :], (2 * m, m)).reshape(_P, 1)
    dr = cc * jnp.cos(theta) / (X * Y)
    di = cc * jnp.sin(theta) / (X * Y)
    pad = S_pad - X * Y
    if pad:
        ert = jnp.pad(ert, ((0, pad), (0, 0)))
        eit = jnp.pad(eit, ((0, pad), (0, 0)))
        dr = jnp.pad(dr, ((0, 0), (0, pad)))
        di = jnp.pad(di, ((0, 0), (0, pad)))
    ertit = jnp.concatenate([ert, eit], axis=1)
    drdi = jnp.concatenate([dr, -di], axis=0)
    return ertit, drdi


def _pack_spec(w1r, w1i, w2r, w2i):
    def flat(w):
        ci, co, m1, m2 = w.shape
        return w.reshape(ci, co, m1 * m2)
    wr = jnp.concatenate([flat(w1r), flat(w2r)], axis=-1)
    wi = jnp.concatenate([flat(w1i), flat(w2i)], axis=-1)
    return wr, wi


def _fno_kernel(G, C, Sp, xa_ref, wl_ref, ertit_ref, drdi_ref,
                wr0_ref, wi0_ref, ws0_ref, wr1_ref, wi1_ref, ws1_ref,
                wr2_ref, wi2_ref, ws2_ref,
                cwx0_ref, cwx1_ref, cwx2_ref,
                f1w_ref, f2w_ref, f2b_ref, o_ref, hs_ref):
    f32 = jnp.float32
    bf16 = jnp.bfloat16
    R = G * C
    E = R + 8

    band = jax.lax.broadcasted_iota(jnp.int32, (8, Sp), 0) == 0
    hs_ref[R:E, :] = band.astype(bf16)
    hs_ref[E:, :] = drdi_ref[...]

    hs_ref[0:R, :] = jax.lax.dot_general(
        wl_ref[...], xa_ref[0], (((1,), (0,)), ((), ())),
        preferred_element_type=f32).astype(bf16)

    wrs = (wr0_ref, wr1_ref, wr2_ref)
    wis = (wi0_ref, wi1_ref, wi2_ref)
    wss = (ws0_ref, ws1_ref, ws2_ref)
    cwxs = (cwx0_ref, cwx1_ref, cwx2_ref)
    NS = 2
    SC = Sp // NS
    for k in range(3):
        xf = jnp.dot(hs_ref[0:R, :], ertit_ref[...],
                     preferred_element_type=f32)
        xr = xf[:, :_P].reshape(G, C, 1, _P)
        xi = xf[:, _P:].reshape(G, C, 1, _P)
        t1 = jnp.sum(xr * wrs[k][...][None], axis=1)
        t2 = jnp.sum(xi * wis[k][...][None], axis=1)
        t3 = jnp.sum((xr + xi) * wss[k][...][None], axis=1)
        omr = t1 - t2
        omi = t3 - t1 - t2
        om = jnp.concatenate([omr.reshape(R, _P), omi.reshape(R, _P)],
                             axis=1).astype(bf16)
        lhs = jnp.concatenate([cwxs[k][...], om], axis=1)
        for sc in range(NS):
            y = jnp.dot(lhs, hs_ref[:, sc * SC:(sc + 1) * SC],
                        preferred_element_type=f32)
            hs_ref[0:R, sc * SC:(sc + 1) * SC] = _gelu_scaled_bf16(y)

    for sc in range(NS):
        t = jnp.dot(f1w_ref[...], hs_ref[0:E, sc * SC:(sc + 1) * SC],
                    preferred_element_type=f32)
        h2 = _gelu_scaled_bf16(t)
        out = jnp.dot(f2w_ref[...], h2, preferred_element_type=f32)
        o_ref[0, :, sc * SC:(sc + 1) * SC] = out + f2b_ref[...]


def kernel(x, fc0_w, fc0_b, conv0_w1r, conv0_w1i, conv0_w2r, conv0_w2i,
           w0_w, w0_b, conv1_w1r, conv1_w1i, conv1_w2r, conv1_w2i, w1_w,
           w1_b, conv2_w1r, conv2_w1i, conv2_w2r, conv2_w2i, w2_w, w2_b,
           fc1_w, fc1_b, fc2_w, fc2_b):
    B, X, Y, Tin = x.shape
    C = _WIDTH
    S = X * Y
    Sp = pl.cdiv(S, 128) * 128
    G = 4 if B % 4 == 0 else (2 if B % 2 == 0 else 1)
    NB = B // G
    f32 = jnp.float32
    bf16 = jnp.bfloat16

    x_cf = jnp.transpose(x, (0, 3, 1, 2)).reshape(B, Tin, S)
    gx = jnp.linspace(-1.5, 1.5, X, dtype=f32)
    gy = jnp.linspace(-2.0, 2.0, Y, dtype=f32)
    grid2 = jnp.stack([jnp.broadcast_to(gx[:, None], (X, Y)).reshape(S),
                       jnp.broadcast_to(gy[None, :], (X, Y)).reshape(S)], 0)
    aug = jnp.concatenate(
        [x_cf,
         jnp.broadcast_to(grid2[None], (B, 2, S)),
         jnp.ones((B, 1, S), f32)], axis=1)
    if Sp != S:
        aug = jnp.pad(aug, ((0, 0), (0, 0), (0, Sp - S)))
    K0 = Tin + 3
    xa = aug.reshape(NB, G * K0, Sp).astype(bf16)

    SQ2 = 1.4142135623730951
    R = G * C
    eyeG = jnp.eye(G, dtype=f32)
    wl = jnp.concatenate([fc0_w.T, fc0_b[:, None]], axis=1) * SQ2
    wl_bd = jnp.kron(eyeG, wl).astype(bf16)

    ertit, drdi = _dft_bases(X, Y, Sp)
    ertit = ertit.astype(bf16)
    drdi = (0.5 * drdi).astype(bf16)

    wr0, wi0 = _pack_spec(conv0_w1r, conv0_w1i, conv0_w2r, conv0_w2i)
    wr1, wi1 = _pack_spec(conv1_w1r, conv1_w1i, conv1_w2r, conv1_w2i)
    wr2, wi2 = _pack_spec(conv2_w1r, conv2_w1i, conv2_w2r, conv2_w2i)
    ws0 = wr0 + wi0
    ws1 = wr1 + wi1
    ws2 = wr2 + wi2

    def conv_ext(w, b):
        bd = jnp.kron(eyeG, 0.5 * w.T)
        col = _INV_SQRT2 * jnp.tile(b, G)[:, None]
        return jnp.concatenate([bd, col, jnp.zeros((R, 7), f32)],
                               axis=1).astype(bf16)

    cwx0 = conv_ext(w0_w, w0_b)
    cwx1 = conv_ext(w1_w, w1_b)
    cwx2 = conv_ext(w2_w, w2_b)

    H = fc1_w.shape[1]
    O = fc2_w.shape[1]
    f1_bd = jnp.kron(eyeG, 0.5 * fc1_w.T)
    f1_col = _INV_SQRT2 * jnp.tile(fc1_b, G)[:, None]
    f1w_ext = jnp.concatenate([f1_bd, f1_col, jnp.zeros((G * H, 7), f32)],
                              axis=1).astype(bf16)
    f2w_bd = jnp.kron(eyeG, _INV_SQRT2 * fc2_w.T).astype(bf16)
    f2b = jnp.tile(fc2_b, G)[:, None]

    const = lambda shp: pl.BlockSpec(shp, lambda i: tuple(0 for _ in shp))
    specP = (C, C, _P)
    out_cf = pl.pallas_call(
        functools.partial(_fno_kernel, G, C, Sp),
        grid=(NB,),
        in_specs=[pl.BlockSpec((1, G * K0, Sp), lambda i: (i, 0, 0)),
                  const((G * C, G * K0)),
                  const((Sp, 2 * _P)), const((2 * _P, Sp)),
                  const(specP), const(specP), const(specP),
                  const(specP), const(specP), const(specP),
                  const(specP), const(specP), const(specP),
                  const((R, R + 8)), const((R, R + 8)), const((R, R + 8)),
                  const((G * H, R + 8)),
                  const((G * O, G * H)), const((G * O, 1))],
        out_specs=pl.BlockSpec((1, G * O, Sp), lambda i: (i, 0, 0)),
        out_shape=jax.ShapeDtypeStruct((NB, G * O, Sp), f32),
        scratch_shapes=[pltpu.VMEM((R + 8 + 2 * _P, Sp), bf16)],
        compiler_params=pltpu.CompilerParams(
            dimension_semantics=("parallel",),
            vmem_limit_bytes=_VMEM_LIMIT),
    )(xa, wl_bd, ertit, drdi,
      wr0, wi0, ws0, wr1, wi1, ws1, wr2, wi2, ws2,
      cwx0, cwx1, cwx2, f1w_ext, f2w_bd, f2b)

    out_cf = out_cf.reshape(B, O, Sp)[:, :, :S]
    return jnp.transpose(out_cf.reshape(B, O, X, Y), (0, 2, 3, 1))


# speedup vs baseline: 1.0372x; 1.0005x over previous
import functools

import jax
import jax.numpy as jnp
from jax.experimental import pallas as pl
from jax.experimental.pallas import tpu as pltpu

_T_IN = 10
_STEP = 10
_MODES = 8
_WIDTH = 32
_P = 2 * _MODES * _MODES
_INV_SQRT2 = 0.7071067811865476
_VMEM_LIMIT = 48 * 1024 * 1024


def _gelu_scaled_bf16(yp):
    return (yp * (1.0 + jax.lax.erf(yp))).astype(jnp.bfloat16)


def _dft_bases(X, Y, S_pad):
    m = _MODES
    kx = jnp.concatenate([jnp.arange(m), jnp.arange(X - m, X)]).astype(jnp.float32)
    ky = jnp.arange(m, dtype=jnp.float32)
    xs = jnp.arange(X, dtype=jnp.float32)
    ys = jnp.arange(Y, dtype=jnp.float32)
    tx = 2.0 * jnp.pi * kx[:, None] * xs[None, :] / X
    ty = 2.0 * jnp.pi * ky[:, None] * ys[None, :] / Y
    theta = (tx[:, None, :, None] + ty[None, :, None, :]).reshape(_P, X * Y)
    ert = jnp.cos(theta).T
    eit = (-jnp.sin(theta)).T
    c = jnp.where(ky == 0, 1.0, 2.0)
    cc = jnp.broadcast_to(c[None, :], (2 * m, m)).reshape(_P, 1)
    dr = cc * jnp.cos(theta) / (X * Y)
    di = cc * jnp.sin(theta) / (X * Y)
    pad = S_pad - X * Y
    if pad:
        ert = jnp.pad(ert, ((0, pad), (0, 0)))
        eit = jnp.pad(eit, ((0, pad), (0, 0)))
        dr = jnp.pad(dr, ((0, 0), (0, pad)))
        di = jnp.pad(di, ((0, 0), (0, pad)))
    ertit = jnp.concatenate([ert, eit], axis=1)
    drdi = jnp.concatenate([dr, -di], axis=0)
    return ertit, drdi


def _pack_spec(w1r, w1i, w2r, w2i):
    def flat(w):
        ci, co, m1, m2 = w.shape
        return w.reshape(ci, co, m1 * m2)
    wr = jnp.concatenate([flat(w1r), flat(w2r)], axis=-1)
    wi = jnp.concatenate([flat(w1i), flat(w2i)], axis=-1)
    return wr, wi


def _fno_kernel(G, C, Sp, xa_ref, wl_ref, ertit_ref, drdi_ref,
                wr0_ref, wi0_ref, ws0_ref, wr1_ref, wi1_ref, ws1_ref,
                wr2_ref, wi2_ref, ws2_ref,
                cwx0_ref, cwx1_ref, cwx2_ref,
                f1w_ref, f2w_ref, f2b_ref, o_ref, hs_ref):
    f32 = jnp.float32
    bf16 = jnp.bfloat16
    R = G * C
    E = R + 8

    band = jax.lax.broadcasted_iota(jnp.int32, (8, Sp), 0) == 0
    hs_ref[R:E, :] = band.astype(bf16)
    hs_ref[E:, :] = drdi_ref[...]

    hs_ref[0:R, :] = jax.lax.dot_general(
        wl_ref[...], xa_ref[0], (((1,), (0,)), ((), ())),
        preferred_element_type=f32).astype(bf16)

    wrs = (wr0_ref, wr1_ref, wr2_ref)
    wis = (wi0_ref, wi1_ref, wi2_ref)
    wss = (ws0_ref, ws1_ref, ws2_ref)
    cwxs = (cwx0_ref, cwx1_ref, cwx2_ref)
    NS = 2
    SC = Sp // NS
    for k in range(3):
        xf = jnp.dot(hs_ref[0:R, :], ertit_ref[...],
                     preferred_element_type=f32)
        xr = xf[:, :_P].reshape(G, C, 1, _P)
        xi = xf[:, _P:].reshape(G, C, 1, _P)
        t1 = jnp.sum(xr * wrs[k][...][None], axis=1)
        t2 = jnp.sum(xi * wis[k][...][None], axis=1)
        t3 = jnp.sum((xr + xi) * wss[k][...][None], axis=1)
        omr = t1 - t2
        omi = t3 - t1 - t2
        om = jnp.concatenate([omr.reshape(R, _P), omi.reshape(R, _P)],
                             axis=1).astype(bf16)
        lhs = jnp.concatenate([cwxs[k][...], om], axis=1)
        for sc in range(NS):
            y = jnp.dot(lhs, hs_ref[:, sc * SC:(sc + 1) * SC],
                        preferred_element_type=f32)
            hs_ref[0:R, sc * SC:(sc + 1) * SC] = _gelu_scaled_bf16(y)

    for sc in range(NS):
        t = jnp.dot(f1w_ref[...], hs_ref[0:E, sc * SC:(sc + 1) * SC],
                    preferred_element_type=f32)
        h2 = _gelu_scaled_bf16(t)
        out = jnp.dot(f2w_ref[...], h2, preferred_element_type=f32)
        o_ref[0, :, sc * SC:(sc + 1) * SC] = out + f2b_ref[...]


def kernel(x, fc0_w, fc0_b, conv0_w1r, conv0_w1i, conv0_w2r, conv0_w2i,
           w0_w, w0_b, conv1_w1r, conv1_w1i, conv1_w2r, conv1_w2i, w1_w,
           w1_b, conv2_w1r, conv2_w1i, conv2_w2r, conv2_w2i, w2_w, w2_b,
           fc1_w, fc1_b, fc2_w, fc2_b):
    B, X, Y, Tin = x.shape
    C = _WIDTH
    S = X * Y
    Sp = pl.cdiv(S, 128) * 128
    G = 4 if B % 4 == 0 else (2 if B % 2 == 0 else 1)
    NB = B // G
    f32 = jnp.float32
    bf16 = jnp.bfloat16

    x_cf = jnp.transpose(x.astype(bf16), (0, 3, 1, 2)).reshape(B, Tin, S)
    gx = jnp.linspace(-1.5, 1.5, X, dtype=f32)
    gy = jnp.linspace(-2.0, 2.0, Y, dtype=f32)
    grid2 = jnp.stack([jnp.broadcast_to(gx[:, None], (X, Y)).reshape(S),
                       jnp.broadcast_to(gy[None, :], (X, Y)).reshape(S)], 0)
    aug = jnp.concatenate(
        [x_cf,
         jnp.broadcast_to(grid2[None].astype(bf16), (B, 2, S)),
         jnp.ones((B, 1, S), bf16)], axis=1)
    if Sp != S:
        aug = jnp.pad(aug, ((0, 0), (0, 0), (0, Sp - S)))
    K0 = Tin + 3
    xa = aug.reshape(NB, G * K0, Sp)

    SQ2 = 1.4142135623730951
    R = G * C
    eyeG = jnp.eye(G, dtype=f32)
    wl = jnp.concatenate([fc0_w.T, fc0_b[:, None]], axis=1) * SQ2
    wl_bd = jnp.kron(eyeG, wl).astype(bf16)

    ertit, drdi = _dft_bases(X, Y, Sp)
    ertit = ertit.astype(bf16)
    drdi = (0.5 * drdi).astype(bf16)

    wr0, wi0 = _pack_spec(conv0_w1r, conv0_w1i, conv0_w2r, conv0_w2i)
    wr1, wi1 = _pack_spec(conv1_w1r, conv1_w1i, conv1_w2r, conv1_w2i)
    wr2, wi2 = _pack_spec(conv2_w1r, conv2_w1i, conv2_w2r, conv2_w2i)
    ws0 = wr0 + wi0
    ws1 = wr1 + wi1
    ws2 = wr2 + wi2

    def conv_ext(w, b):
        bd = jnp.kron(eyeG, 0.5 * w.T)
        col = _INV_SQRT2 * jnp.tile(b, G)[:, None]
        return jnp.concatenate([bd, col, jnp.zeros((R, 7), f32)],
                               axis=1).astype(bf16)

    cwx0 = conv_ext(w0_w, w0_b)
    cwx1 = conv_ext(w1_w, w1_b)
    cwx2 = conv_ext(w2_w, w2_b)

    H = fc1_w.shape[1]
    O = fc2_w.shape[1]
    f1_bd = jnp.kron(eyeG, 0.5 * fc1_w.T)
    f1_col = _INV_SQRT2 * jnp.tile(fc1_b, G)[:, None]
    f1w_ext = jnp.concatenate([f1_bd, f1_col, jnp.zeros((G * H, 7), f32)],
                              axis=1).astype(bf16)
    f2w_bd = jnp.kron(eyeG, _INV_SQRT2 * fc2_w.T).astype(bf16)
    f2b = jnp.tile(fc2_b, G)[:, None]

    const = lambda shp: pl.BlockSpec(shp, lambda i: tuple(0 for _ in shp))
    specP = (C, C, _P)
    out_cf = pl.pallas_call(
        functools.partial(_fno_kernel, G, C, Sp),
        grid=(NB,),
        in_specs=[pl.BlockSpec((1, G * K0, Sp), lambda i: (i, 0, 0)),
                  const((G * C, G * K0)),
                  const((Sp, 2 * _P)), const((2 * _P, Sp)),
                  const(specP), const(specP), const(specP),
                  const(specP), const(specP), const(specP),
                  const(specP), const(specP), const(specP),
                  const((R, R + 8)), const((R, R + 8)), const((R, R + 8)),
                  const((G * H, R + 8)),
                  const((G * O, G * H)), const((G * O, 1))],
        out_specs=pl.BlockSpec((1, G * O, Sp), lambda i: (i, 0, 0)),
        out_shape=jax.ShapeDtypeStruct((NB, G * O, Sp), f32),
        scratch_shapes=[pltpu.VMEM((R + 8 + 2 * _P, Sp), bf16)],
        compiler_params=pltpu.CompilerParams(
            dimension_semantics=("parallel",),
            vmem_limit_bytes=_VMEM_LIMIT),
    )(xa, wl_bd, ertit, drdi,
      wr0, wi0, ws0, wr1, wi1, ws1, wr2, wi2, ws2,
      cwx0, cwx1, cwx2, f1w_ext, f2w_bd, f2b)

    out_cf = out_cf.reshape(B, O, Sp)[:, :, :S]
    return jnp.transpose(out_cf.reshape(B, O, X, Y), (0, 2, 3, 1))
